# Initial kernel scaffold; baseline (speedup 1.0000x reference)
import jax
import jax.numpy as jnp
from jax import lax
from jax.experimental import pallas as pl
from jax.experimental.pallas import tpu as pltpu
from jax.experimental.pallas import tpu_sc as plsc

D = 128
E = 320000
NC, NS, L = 2, 16, 16
NW = NC * NS
EW = E // NW
C = 80
K = EW // C
G = C // L


def _body(z_hbm, src_hbm, dst_hbm, out_hbm,
          idx_s, idx_d, srows, drows, out_buf, sem_s, sem_d):
    wid = lax.axis_index("c") * NS + lax.axis_index("s")
    base = wid * EW
    iota = lax.iota(jnp.int32, L)

    def chunk_body(c, carry):
        off = base + c * C
        pltpu.sync_copy(src_hbm.at[pl.ds(off, C)], idx_s)
        pltpu.sync_copy(dst_hbm.at[pl.ds(off, C)], idx_d)
        cp_s = pltpu.async_copy(z_hbm.at[idx_s], srows, sem_s)
        cp_d = pltpu.async_copy(z_hbm.at[idx_d], drows, sem_d)
        cp_s.wait()
        cp_d.wait()

        def group_body(g, carry2):
            rows = g * L + iota
            acc = jnp.zeros((L,), jnp.float32)
            for j in range(D):
                col = jnp.full((L,), j, jnp.int32)
                vs = plsc.load_gather(srows, [rows, col])
                vd = plsc.load_gather(drows, [rows, col])
                acc = acc + vs * vd
            out = 1.0 / (1.0 + jnp.exp(-acc))
            out_buf[pl.ds(c * C + g * L, L)] = out
            return carry2

        lax.fori_loop(0, G, group_body, 0, unroll=False)
        return carry

    lax.fori_loop(0, K, chunk_body, 0, unroll=False)
    pltpu.sync_copy(out_buf, out_hbm.at[pl.ds(base, EW)])


@jax.jit
def _gae_decode(z, src, dst):
    mesh = plsc.VectorSubcoreMesh(core_axis_name="c", subcore_axis_name="s")
    return pl.kernel(
        _body,
        out_type=jax.ShapeDtypeStruct((E,), jnp.float32),
        mesh=mesh,
        scratch_types=[
            pltpu.VMEM((C,), jnp.int32),
            pltpu.VMEM((C,), jnp.int32),
            pltpu.VMEM((C, D), jnp.float32),
            pltpu.VMEM((C, D), jnp.float32),
            pltpu.VMEM((EW,), jnp.float32),
            pltpu.SemaphoreType.DMA,
            pltpu.SemaphoreType.DMA,
        ],
    )(z, src, dst)


def kernel(z, edge_index):
    src = edge_index[0].astype(jnp.int32)
    dst = edge_index[1].astype(jnp.int32)
    return _gae_decode(z, src, dst)


# speedup vs baseline: 1.1015x; 1.1015x over previous
import jax
import jax.numpy as jnp
from jax import lax
from jax.experimental import pallas as pl
from jax.experimental.pallas import tpu as pltpu
from jax.experimental.pallas import tpu_sc as plsc

D = 128
E = 320000
NC, NS, L = 2, 16, 16
NW = NC * NS
EW = E // NW
C = 80
K = EW // C
G = C // L


def _body(z_hbm, src_hbm, dst_hbm, out_hbm,
          idx_s, idx_d, srows, drows, out_buf, sem_s, sem_d):
    wid = lax.axis_index("c") * NS + lax.axis_index("s")
    base = wid * EW
    iota = lax.iota(jnp.int32, L)

    def chunk_body(c, carry):
        off = base + c * C
        pltpu.sync_copy(src_hbm.at[pl.ds(off, C)], idx_s)
        pltpu.sync_copy(dst_hbm.at[pl.ds(off, C)], idx_d)
        cp_s = pltpu.async_copy(z_hbm.at[idx_s], srows, sem_s)
        cp_d = pltpu.async_copy(z_hbm.at[idx_d], drows, sem_d)
        cp_s.wait()
        cp_d.wait()

        def group_body(g, carry2):
            rows = g * L + iota
            acc = jnp.zeros((L,), jnp.float32)
            for j in range(D):
                col = jnp.full((L,), j, jnp.int32)
                vs = plsc.load_gather(srows, [rows, col])
                vd = plsc.load_gather(drows, [rows, col])
                acc = acc + vs * vd
            out = 1.0 / (1.0 + jnp.exp(-acc))
            out_buf[pl.ds(c * C + g * L, L)] = out
            return carry2

        lax.fori_loop(0, G, group_body, 0, unroll=False)
        return carry

    lax.fori_loop(0, K, chunk_body, 0, unroll=False)
    pltpu.sync_copy(out_buf, out_hbm.at[pl.ds(base, EW)])


@jax.jit
def _gae_decode(z, src, dst):
    mesh = plsc.VectorSubcoreMesh(core_axis_name="c", subcore_axis_name="s")
    return pl.kernel(
        _body,
        out_type=jax.ShapeDtypeStruct((E,), jnp.float32),
        mesh=mesh,
        compiler_params=pltpu.CompilerParams(needs_layout_passes=False),
        scratch_types=[
            pltpu.VMEM((C,), jnp.int32),
            pltpu.VMEM((C,), jnp.int32),
            pltpu.VMEM((C, D), jnp.float32),
            pltpu.VMEM((C, D), jnp.float32),
            pltpu.VMEM((EW,), jnp.float32),
            pltpu.SemaphoreType.DMA,
            pltpu.SemaphoreType.DMA,
        ],
    )(z, src, dst)


def kernel(z, edge_index):
    src = edge_index[0].astype(jnp.int32)
    dst = edge_index[1].astype(jnp.int32)
    return _gae_decode(z, src, dst)


# speedup vs baseline: 8.0024x; 7.2647x over previous
import jax
import jax.numpy as jnp
from jax import lax
from jax.experimental import pallas as pl
from jax.experimental.pallas import tpu as pltpu
from jax.experimental.pallas import tpu_sc as plsc

D = 128
E = 320000
NC, NS, L = 2, 16, 16
NW = NC * NS
EW = E // NW
C = 80
K = EW // C
G = C // L
PS = L + 1


def _body(z_hbm, src_hbm, dst_hbm, out_hbm,
          idx_s, idx_d, srows0, drows0, srows1, drows1, p1, out_buf,
          sem_s0, sem_d0, sem_s1, sem_d1):
    wid = lax.axis_index("c") * NS + lax.axis_index("s")
    base = wid * EW
    iota = lax.iota(jnp.int32, L)
    iota_ps = iota * PS

    slots = ((srows0, drows0, sem_s0, sem_d0),
             (srows1, drows1, sem_s1, sem_d1))

    pltpu.sync_copy(src_hbm.at[pl.ds(base, EW)], idx_s)
    pltpu.sync_copy(dst_hbm.at[pl.ds(base, EW)], idx_d)

    def issue(c, slot):
        sr, dr, ss, sd = slots[slot]
        pltpu.async_copy(z_hbm.at[idx_s.at[pl.ds(c * C, C)]], sr, ss)
        pltpu.async_copy(z_hbm.at[idx_d.at[pl.ds(c * C, C)]], dr, sd)

    def wait(c, slot):
        sr, dr, ss, sd = slots[slot]
        pltpu.make_async_copy(z_hbm.at[idx_s.at[pl.ds(c * C, C)]], sr, ss).wait()
        pltpu.make_async_copy(z_hbm.at[idx_d.at[pl.ds(c * C, C)]], dr, sd).wait()

    def compute(c, slot):
        sr, dr, _, _ = slots[slot]

        def group_body(g, carry):
            for i in range(L):
                e = g * L + i
                p = None
                for k in range(D // L):
                    vs = sr[e, pl.ds(k * L, L)]
                    vd = dr[e, pl.ds(k * L, L)]
                    prod = vs * vd
                    p = prod if p is None else p + prod
                p1[pl.ds(i * PS, L)] = p
            acc = plsc.load_gather(p1, [iota_ps])
            for j in range(1, L):
                acc = acc + plsc.load_gather(p1, [iota_ps + j])
            out = 1.0 / (1.0 + jnp.exp(-acc))
            out_buf[pl.ds(c * C + g * L, L)] = out
            return carry

        lax.fori_loop(0, G, group_body, 0, unroll=False)

    issue(0, 0)
    issue(1, 1)

    def pair_body(i, carry):
        c0 = 2 * i
        wait(c0, 0)
        compute(c0, 0)
        issue(c0 + 2, 0)

        c1 = 2 * i + 1
        wait(c1, 1)
        compute(c1, 1)

        @pl.when(c1 + 2 < K)
        def _():
            issue(c1 + 2, 1)

        return carry

    lax.fori_loop(0, K // 2, pair_body, 0, unroll=False)
    wait(K - 1, 0)
    compute(K - 1, 0)

    pltpu.sync_copy(out_buf, out_hbm.at[pl.ds(base, EW)])


@jax.jit
def _gae_decode(z, src, dst):
    mesh = plsc.VectorSubcoreMesh(core_axis_name="c", subcore_axis_name="s")
    return pl.kernel(
        _body,
        out_type=jax.ShapeDtypeStruct((E,), jnp.float32),
        mesh=mesh,
        compiler_params=pltpu.CompilerParams(needs_layout_passes=False),
        scratch_types=[
            pltpu.VMEM((EW,), jnp.int32),
            pltpu.VMEM((EW,), jnp.int32),
            pltpu.VMEM((C, D), jnp.float32),
            pltpu.VMEM((C, D), jnp.float32),
            pltpu.VMEM((C, D), jnp.float32),
            pltpu.VMEM((C, D), jnp.float32),
            pltpu.VMEM((L * PS,), jnp.float32),
            pltpu.VMEM((EW,), jnp.float32),
            pltpu.SemaphoreType.DMA,
            pltpu.SemaphoreType.DMA,
            pltpu.SemaphoreType.DMA,
            pltpu.SemaphoreType.DMA,
        ],
    )(z, src, dst)


def kernel(z, edge_index):
    src = edge_index[0].astype(jnp.int32)
    dst = edge_index[1].astype(jnp.int32)
    return _gae_decode(z, src, dst)
